# baseline (speedup 1.0000x reference)
import functools

import jax
import jax.numpy as jnp
from jax import lax
from jax.experimental import pallas as pl
from jax.experimental.pallas import tpu as pltpu
from jax.experimental.pallas import tpu_sc as plsc

NN = 50000
EE = 800000
DIN = 768
DH = 64
DO = 32

LANE = 128
EROWS = 6400
EPAD = EROWS * LANE
RB = 4096
NP = 53248
NCORE = 2
NSUB = 16
NB = NP // RB
IB = 16
OUTER = EROWS // NSUB // IB
DIB = 20
DOUTER = EROWS // (NCORE * NSUB) // DIB

_mesh = plsc.VectorSubcoreMesh(core_axis_name="c", subcore_axis_name="s")
_sc_params = pltpu.CompilerParams(use_tc_tiling_on_sc=False)


DEGW = 16


def _deg_body(dst_hbm, zeros_hbm, ones_hbm, out_hbm, dst_bufs, ones_buf, acc,
              isem, ssem):
    cid = lax.axis_index("c")
    sid = lax.axis_index("s")
    zr = NP // NSUB
    pltpu.sync_copy(zeros_hbm.at[pl.ds(sid * zr, zr)], acc.at[pl.ds(sid * zr, zr)])
    pltpu.sync_copy(ones_hbm, ones_buf)
    plsc.subcore_barrier()
    rows_per = EROWS // (NCORE * NSUB)
    base = cid * (EROWS // NCORE) + sid * rows_per
    pltpu.sync_copy(dst_hbm.at[pl.ds(base, DIB)], dst_bufs.at[pl.ds(0, DIB)])

    @pl.loop(0, DOUTER)
    def _(g):
        @pl.when(g >= 1)
        def _():
            @pl.loop(0, DIB)
            def _(i):
                pltpu.make_async_copy(ones_hbm, ones_buf, ssem).wait()

        @pl.when(g < DOUTER - 1)
        def _():
            pltpu.async_copy(
                dst_hbm.at[pl.ds(base + (g + 1) * DIB, DIB)],
                dst_bufs.at[pl.ds(((g + 1) % 2) * DIB, DIB)], isem)
        for j in range(DIB):
            pltpu.async_copy(ones_buf, acc.at[dst_bufs.at[(g % 2) * DIB + j]],
                             ssem, add=True)

        @pl.when(g < DOUTER - 1)
        def _():
            pltpu.make_async_copy(
                dst_hbm.at[pl.ds(base, DIB)],
                dst_bufs.at[pl.ds(0, DIB)], isem).wait()

    @pl.loop(0, DIB)
    def _(i):
        pltpu.make_async_copy(ones_hbm, ones_buf, ssem).wait()

    plsc.subcore_barrier()
    pltpu.sync_copy(acc.at[pl.ds(sid * zr, zr)],
                    out_hbm.at[pl.ds(cid * NP + sid * zr, zr)])


_deg_call = pl.kernel(
    _deg_body,
    out_type=jax.ShapeDtypeStruct((2 * NP, DEGW), jnp.float32),
    mesh=_mesh,
    compiler_params=_sc_params,
    scratch_types=[
        pltpu.VMEM((2 * DIB, LANE), jnp.int32),
        pltpu.VMEM((LANE, DEGW), jnp.float32),
        pltpu.VMEM_SHARED((NP, DEGW), jnp.float32),
        pltpu.SemaphoreType.DMA,
        pltpu.SemaphoreType.DMA,
    ],
)


def _agg_body(width, spm_table, tab_a, tab_b, src_hbm, dst_hbm, zeros_hbm,
              out_hbm, src_bufs, dst_bufs, rows, acc, tabspm,
              gsems, ssems, isem_s, isem_d):
    cid = lax.axis_index("c")
    sid = lax.axis_index("s")
    zr = NP // NSUB
    pltpu.sync_copy(zeros_hbm.at[pl.ds(sid * zr, zr)], acc.at[pl.ds(sid * zr, zr)])
    if spm_table:
        @pl.when(cid == 0)
        def _():
            pltpu.sync_copy(tab_a.at[pl.ds(sid * zr, zr)],
                            tabspm.at[pl.ds(sid * zr, zr)])

        @pl.when(cid == 1)
        def _():
            pltpu.sync_copy(tab_b.at[pl.ds(sid * zr, zr)],
                            tabspm.at[pl.ds(sid * zr, zr)])
    rows_per = EROWS // NSUB
    ebase = sid * rows_per

    def wait_scatter(b):
        pltpu.make_async_copy(tab_a.at[pl.ds(0, LANE)], rows.at[b],
                              ssems.at[b]).wait()

    def wait_gather(b):
        pltpu.make_async_copy(tab_a.at[pl.ds(0, LANE)], rows.at[b],
                              gsems.at[b]).wait()

    if spm_table:
        def start_gather(idx_row, b):
            pltpu.async_copy(tabspm.at[src_bufs.at[idx_row]], rows.at[b],
                             gsems.at[b])
    else:
        def start_gather(idx_row, b):
            @pl.when(cid == 0)
            def _():
                pltpu.async_copy(tab_a.at[src_bufs.at[idx_row]], rows.at[b],
                                 gsems.at[b])

            @pl.when(cid == 1)
            def _():
                pltpu.async_copy(tab_b.at[src_bufs.at[idx_row]], rows.at[b],
                                 gsems.at[b])

    pltpu.sync_copy(src_hbm.at[pl.ds(ebase, IB)], src_bufs.at[pl.ds(0, IB)])
    pltpu.sync_copy(dst_hbm.at[pl.ds(ebase, IB)], dst_bufs.at[pl.ds(0, IB)])
    plsc.subcore_barrier()
    for k in range(3):
        start_gather(k, k)

    @pl.loop(0, OUTER)
    def _(g):
        gb = (g % 2) * IB
        nb = ((g + 1) % 2) * IB
        for j in range(IB):
            b = j % 4
            b3 = (j + 3) % 4
            wait_gather(b)
            pltpu.async_copy(rows.at[b], acc.at[dst_bufs.at[gb + j]],
                             ssems.at[b], add=True)
            if j == 1:
                @pl.when(g < OUTER - 1)
                def _():
                    pltpu.async_copy(src_hbm.at[pl.ds(ebase + (g + 1) * IB, IB)],
                                     src_bufs.at[pl.ds(nb, IB)], isem_s)
                    pltpu.async_copy(dst_hbm.at[pl.ds(ebase + (g + 1) * IB, IB)],
                                     dst_bufs.at[pl.ds(nb, IB)], isem_d)
            if j < IB - 3:
                if j == 0:
                    @pl.when(g >= 1)
                    def _():
                        wait_scatter(b3)
                else:
                    wait_scatter(b3)
                start_gather(gb + j + 3, b3)
            else:
                if j == IB - 3:
                    @pl.when(g < OUTER - 1)
                    def _():
                        pltpu.make_async_copy(src_hbm.at[pl.ds(0, IB)],
                                              src_bufs.at[pl.ds(nb, IB)],
                                              isem_s).wait()
                        pltpu.make_async_copy(dst_hbm.at[pl.ds(0, IB)],
                                              dst_bufs.at[pl.ds(nb, IB)],
                                              isem_d).wait()

                @pl.when(g < OUTER - 1)
                def _():
                    wait_scatter(b3)
                    start_gather(nb + j - (IB - 3), b3)

    for k in range(4):
        wait_scatter(k)
    plsc.subcore_barrier()
    pltpu.sync_copy(acc.at[pl.ds(sid * zr, zr)],
                    out_hbm.at[pl.ds(cid * NP + sid * zr, zr)])


def _make_agg(width, spm_table):
    return pl.kernel(
        functools.partial(_agg_body, width, spm_table),
        out_type=jax.ShapeDtypeStruct((2 * NP, width), jnp.float32),
        mesh=_mesh,
        compiler_params=_sc_params,
        scratch_types=[
            pltpu.VMEM((2 * IB, LANE), jnp.int32),
            pltpu.VMEM((2 * IB, LANE), jnp.int32),
            pltpu.VMEM((4, LANE, width), jnp.float32),
            pltpu.VMEM_SHARED((NP, width), jnp.float32),
            pltpu.VMEM_SHARED((NP, width if spm_table else 1), jnp.float32),
            pltpu.SemaphoreType.DMA((4,)),
            pltpu.SemaphoreType.DMA((4,)),
            pltpu.SemaphoreType.DMA,
            pltpu.SemaphoreType.DMA,
        ],
    )


_agg32_call = _make_agg(DH // 2, False)
_agg16_call = _make_agg(DO // 2, True)


def _mm1_body(x_ref, w_ref, d0_ref, d1_ref, dinv_ref, hsa_ref, hsb_ref):
    deg = d0_ref[:, :1] + d1_ref[:, :1] + 1.0
    dinv = lax.rsqrt(deg)
    dinv_ref[...] = dinv
    hs = jnp.dot(x_ref[...], w_ref[...],
                 preferred_element_type=jnp.float32) * dinv
    hsa_ref[...] = hs[:, :DH // 2]
    hsb_ref[...] = hs[:, DH // 2:]


def _mm1_call(x, w, deg2):
    return pl.pallas_call(
        _mm1_body,
        grid=(NB,),
        in_specs=[
            pl.BlockSpec((RB, DIN), lambda i: (i, 0)),
            pl.BlockSpec((DIN, DH), lambda i: (0, 0)),
            pl.BlockSpec((RB, DEGW), lambda i: (i, 0)),
            pl.BlockSpec((RB, DEGW), lambda i: (NB + i, 0)),
        ],
        out_specs=[
            pl.BlockSpec((RB, 1), lambda i: (i, 0)),
            pl.BlockSpec((RB, DH // 2), lambda i: (i, 0)),
            pl.BlockSpec((RB, DH // 2), lambda i: (i, 0)),
        ],
        out_shape=[
            jax.ShapeDtypeStruct((NP, 1), jnp.float32),
            jax.ShapeDtypeStruct((NP, DH // 2), jnp.float32),
            jax.ShapeDtypeStruct((NP, DH // 2), jnp.float32),
        ],
    )(x, w, deg2, deg2)


def _mid_body(aga_ref, agb_ref, hsa_ref, hsb_ref, dinv_ref, b1_ref, w2_ref,
              h2a_ref, h2b_ref):
    dinv = dinv_ref[...]
    a = jnp.concatenate([aga_ref[...] + hsa_ref[...],
                         agb_ref[...] + hsb_ref[...]], axis=1)
    h = jnp.maximum(dinv * a + b1_ref[...], 0.0)
    p2 = jnp.dot(h, w2_ref[...], preferred_element_type=jnp.float32)
    hs2 = dinv * p2
    h2a_ref[...] = hs2[:, :DO // 2]
    h2b_ref[...] = hs2[:, DO // 2:]


def _mid_call(agg1, hs1a, hs1b, dinv, b1, w2):
    return pl.pallas_call(
        _mid_body,
        grid=(NB,),
        in_specs=[
            pl.BlockSpec((RB, DH // 2), lambda i: (i, 0)),
            pl.BlockSpec((RB, DH // 2), lambda i: (NB + i, 0)),
            pl.BlockSpec((RB, DH // 2), lambda i: (i, 0)),
            pl.BlockSpec((RB, DH // 2), lambda i: (i, 0)),
            pl.BlockSpec((RB, 1), lambda i: (i, 0)),
            pl.BlockSpec((1, DH), lambda i: (0, 0)),
            pl.BlockSpec((DH, DO), lambda i: (0, 0)),
        ],
        out_specs=[
            pl.BlockSpec((RB, DO // 2), lambda i: (i, 0)),
            pl.BlockSpec((RB, DO // 2), lambda i: (i, 0)),
        ],
        out_shape=[
            jax.ShapeDtypeStruct((NP, DO // 2), jnp.float32),
            jax.ShapeDtypeStruct((NP, DO // 2), jnp.float32),
        ],
    )(agg1, agg1, hs1a, hs1b, dinv, b1, w2)


def _out_body(aga_ref, agb_ref, hsa_ref, hsb_ref, dinv_ref, b2_ref, o_ref):
    a = jnp.concatenate([aga_ref[...] + hsa_ref[...],
                         agb_ref[...] + hsb_ref[...]], axis=1)
    o_ref[...] = dinv_ref[...] * a + b2_ref[...]


def _out_call(agg2, hs2a, hs2b, dinv, b2):
    return pl.pallas_call(
        _out_body,
        grid=(NB,),
        in_specs=[
            pl.BlockSpec((RB, DO // 2), lambda i: (i, 0)),
            pl.BlockSpec((RB, DO // 2), lambda i: (NB + i, 0)),
            pl.BlockSpec((RB, DO // 2), lambda i: (i, 0)),
            pl.BlockSpec((RB, DO // 2), lambda i: (i, 0)),
            pl.BlockSpec((RB, 1), lambda i: (i, 0)),
            pl.BlockSpec((1, DO), lambda i: (0, 0)),
        ],
        out_specs=pl.BlockSpec((RB, DO), lambda i: (i, 0)),
        out_shape=jax.ShapeDtypeStruct((NN, DO), jnp.float32),
    )(agg2, agg2, hs2a, hs2b, dinv, b2)


def kernel(x, edge_index, W1, b1, W2, b2):
    src = edge_index[0].astype(jnp.int32)
    dst = edge_index[1].astype(jnp.int32)
    npad = EPAD - EE
    srcI = jnp.concatenate([src, jnp.zeros((npad,), jnp.int32)]).reshape(
        EROWS, LANE)
    dstI = jnp.concatenate([dst, jnp.full((npad,), NN, jnp.int32)]).reshape(
        EROWS, LANE)
    zeros32 = jnp.zeros((NP, DH // 2), jnp.float32)
    zeros16 = jnp.zeros((NP, DO // 2), jnp.float32)
    zerosd = jnp.zeros((NP, DEGW), jnp.float32)
    ones128 = jnp.ones((LANE, DEGW), jnp.float32)

    deg2 = _deg_call(dstI, zerosd, ones128)
    dinv, hs1a, hs1b = _mm1_call(x, W1, deg2)
    agg1 = _agg32_call(hs1a, hs1b, srcI, dstI, zeros32)
    hs2a, hs2b = _mid_call(agg1, hs1a, hs1b, dinv,
                           b1.reshape(1, DH), W2)
    agg2 = _agg16_call(hs2a, hs2b, srcI, dstI, zeros16)
    return _out_call(agg2, hs2a, hs2b, dinv, b2.reshape(1, DO))


# speedup vs baseline: 1.1569x; 1.1569x over previous
import functools

import jax
import jax.numpy as jnp
from jax import lax
from jax.experimental import pallas as pl
from jax.experimental.pallas import tpu as pltpu
from jax.experimental.pallas import tpu_sc as plsc

NN = 50000
EE = 800000
DIN = 768
DH = 64
DO = 32

LANE = 128
EROWS = 6400
EPAD = EROWS * LANE
RB = 4096
NP = 53248
NCORE = 2
NSUB = 16
NB = NP // RB
IB = 16
OUTER = EROWS // NSUB // IB
DIB = 20
DOUTER = EROWS // (NCORE * NSUB) // DIB

_mesh = plsc.VectorSubcoreMesh(core_axis_name="c", subcore_axis_name="s")
_sc_params = pltpu.CompilerParams(use_tc_tiling_on_sc=False)


DEGW = 16


def _deg_body(dst_hbm, zeros_hbm, ones_hbm, out_hbm, dst_bufs, ones_buf, acc,
              isem, ssem):
    cid = lax.axis_index("c")
    sid = lax.axis_index("s")
    zr = NP // NSUB
    pltpu.sync_copy(zeros_hbm.at[pl.ds(sid * zr, zr)], acc.at[pl.ds(sid * zr, zr)])
    pltpu.sync_copy(ones_hbm, ones_buf)
    plsc.subcore_barrier()
    rows_per = EROWS // (NCORE * NSUB)
    base = cid * (EROWS // NCORE) + sid * rows_per
    pltpu.sync_copy(dst_hbm.at[pl.ds(base, DIB)], dst_bufs.at[pl.ds(0, DIB)])

    @pl.loop(0, DOUTER)
    def _(g):
        @pl.when(g >= 1)
        def _():
            @pl.loop(0, DIB)
            def _(i):
                pltpu.make_async_copy(ones_hbm, ones_buf, ssem).wait()

        @pl.when(g < DOUTER - 1)
        def _():
            pltpu.async_copy(
                dst_hbm.at[pl.ds(base + (g + 1) * DIB, DIB)],
                dst_bufs.at[pl.ds(((g + 1) % 2) * DIB, DIB)], isem)
        for j in range(DIB):
            pltpu.async_copy(ones_buf, acc.at[dst_bufs.at[(g % 2) * DIB + j]],
                             ssem, add=True)

        @pl.when(g < DOUTER - 1)
        def _():
            pltpu.make_async_copy(
                dst_hbm.at[pl.ds(base, DIB)],
                dst_bufs.at[pl.ds(0, DIB)], isem).wait()

    @pl.loop(0, DIB)
    def _(i):
        pltpu.make_async_copy(ones_hbm, ones_buf, ssem).wait()

    plsc.subcore_barrier()
    pltpu.sync_copy(acc.at[pl.ds(sid * zr, zr)],
                    out_hbm.at[pl.ds(cid * NP + sid * zr, zr)])


_deg_call = pl.kernel(
    _deg_body,
    out_type=jax.ShapeDtypeStruct((2 * NP, DEGW), jnp.float32),
    mesh=_mesh,
    compiler_params=_sc_params,
    scratch_types=[
        pltpu.VMEM((2 * DIB, LANE), jnp.int32),
        pltpu.VMEM((LANE, DEGW), jnp.float32),
        pltpu.VMEM_SHARED((NP, DEGW), jnp.float32),
        pltpu.SemaphoreType.DMA,
        pltpu.SemaphoreType.DMA,
    ],
)


def _agg_body(width, spm_table, tab_a, tab_b, src_hbm, dst_hbm, zeros_hbm,
              out_hbm, src_bufs, dst_bufs, rows, acc, tabspm,
              gsems, ssems, isem_s, isem_d):
    cid = lax.axis_index("c")
    sid = lax.axis_index("s")
    zr = NP // NSUB
    pltpu.sync_copy(zeros_hbm.at[pl.ds(sid * zr, zr)], acc.at[pl.ds(sid * zr, zr)])
    if spm_table:
        @pl.when(cid == 0)
        def _():
            pltpu.sync_copy(tab_a.at[pl.ds(sid * zr, zr)],
                            tabspm.at[pl.ds(sid * zr, zr)])

        @pl.when(cid == 1)
        def _():
            pltpu.sync_copy(tab_b.at[pl.ds(sid * zr, zr)],
                            tabspm.at[pl.ds(sid * zr, zr)])
    rows_per = EROWS // NSUB
    ebase = sid * rows_per

    def wait_scatter(b):
        pltpu.make_async_copy(tab_a.at[pl.ds(0, LANE)], rows.at[b],
                              ssems.at[b]).wait()

    def wait_gather(b):
        pltpu.make_async_copy(tab_a.at[pl.ds(0, LANE)], rows.at[b],
                              gsems.at[b]).wait()

    if spm_table:
        def start_gather(idx_row, b):
            pltpu.async_copy(tabspm.at[src_bufs.at[idx_row]], rows.at[b],
                             gsems.at[b])
    else:
        def start_gather(idx_row, b):
            @pl.when(cid == 0)
            def _():
                pltpu.async_copy(tab_a.at[src_bufs.at[idx_row]], rows.at[b],
                                 gsems.at[b])

            @pl.when(cid == 1)
            def _():
                pltpu.async_copy(tab_b.at[src_bufs.at[idx_row]], rows.at[b],
                                 gsems.at[b])

    pltpu.sync_copy(src_hbm.at[pl.ds(ebase, IB)], src_bufs.at[pl.ds(0, IB)])
    pltpu.sync_copy(dst_hbm.at[pl.ds(ebase, IB)], dst_bufs.at[pl.ds(0, IB)])
    plsc.subcore_barrier()
    for k in range(3):
        start_gather(k, k)

    @pl.loop(0, OUTER)
    def _(g):
        gb = (g % 2) * IB
        nb = ((g + 1) % 2) * IB
        for j in range(IB):
            b = j % 4
            b3 = (j + 3) % 4
            wait_gather(b)
            pltpu.async_copy(rows.at[b], acc.at[dst_bufs.at[gb + j]],
                             ssems.at[b], add=True)
            if j == 1:
                @pl.when(g < OUTER - 1)
                def _():
                    pltpu.async_copy(src_hbm.at[pl.ds(ebase + (g + 1) * IB, IB)],
                                     src_bufs.at[pl.ds(nb, IB)], isem_s)
                    pltpu.async_copy(dst_hbm.at[pl.ds(ebase + (g + 1) * IB, IB)],
                                     dst_bufs.at[pl.ds(nb, IB)], isem_d)
            if j < IB - 3:
                if j == 0:
                    @pl.when(g >= 1)
                    def _():
                        wait_scatter(b3)
                else:
                    wait_scatter(b3)
                start_gather(gb + j + 3, b3)
            else:
                if j == IB - 3:
                    @pl.when(g < OUTER - 1)
                    def _():
                        pltpu.make_async_copy(src_hbm.at[pl.ds(0, IB)],
                                              src_bufs.at[pl.ds(nb, IB)],
                                              isem_s).wait()
                        pltpu.make_async_copy(dst_hbm.at[pl.ds(0, IB)],
                                              dst_bufs.at[pl.ds(nb, IB)],
                                              isem_d).wait()

                @pl.when(g < OUTER - 1)
                def _():
                    wait_scatter(b3)
                    start_gather(nb + j - (IB - 3), b3)

    for k in range(4):
        wait_scatter(k)
    plsc.subcore_barrier()
    pltpu.sync_copy(acc.at[pl.ds(sid * zr, zr)],
                    out_hbm.at[pl.ds(cid * NP + sid * zr, zr)])


def _make_agg(width, spm_table):
    return pl.kernel(
        functools.partial(_agg_body, width, spm_table),
        out_type=jax.ShapeDtypeStruct((2 * NP, width), jnp.float32),
        mesh=_mesh,
        compiler_params=_sc_params,
        scratch_types=[
            pltpu.VMEM((2 * IB, LANE), jnp.int32),
            pltpu.VMEM((2 * IB, LANE), jnp.int32),
            pltpu.VMEM((4, LANE, width), jnp.float32),
            pltpu.VMEM_SHARED((NP, width), jnp.float32),
            pltpu.VMEM_SHARED((NP, width if spm_table else 1), jnp.float32),
            pltpu.SemaphoreType.DMA((4,)),
            pltpu.SemaphoreType.DMA((4,)),
            pltpu.SemaphoreType.DMA,
            pltpu.SemaphoreType.DMA,
        ],
    )


_aggq_call = _make_agg(16, True)


def _mm1_body(x_ref, w_ref, d0_ref, d1_ref, dinv_ref, q0_ref, q1_ref,
              q2_ref, q3_ref):
    deg = d0_ref[:, :1] + d1_ref[:, :1] + 1.0
    dinv = lax.rsqrt(deg)
    dinv_ref[...] = dinv
    hs = jnp.dot(x_ref[...], w_ref[...],
                 preferred_element_type=jnp.float32) * dinv
    q0_ref[...] = hs[:, 0:16]
    q1_ref[...] = hs[:, 16:32]
    q2_ref[...] = hs[:, 32:48]
    q3_ref[...] = hs[:, 48:64]


def _mm1_call(x, w, deg2):
    return pl.pallas_call(
        _mm1_body,
        grid=(NB,),
        in_specs=[
            pl.BlockSpec((RB, DIN), lambda i: (i, 0)),
            pl.BlockSpec((DIN, DH), lambda i: (0, 0)),
            pl.BlockSpec((RB, DEGW), lambda i: (i, 0)),
            pl.BlockSpec((RB, DEGW), lambda i: (NB + i, 0)),
        ],
        out_specs=[
            pl.BlockSpec((RB, 1), lambda i: (i, 0)),
            pl.BlockSpec((RB, 16), lambda i: (i, 0)),
            pl.BlockSpec((RB, 16), lambda i: (i, 0)),
            pl.BlockSpec((RB, 16), lambda i: (i, 0)),
            pl.BlockSpec((RB, 16), lambda i: (i, 0)),
        ],
        out_shape=[
            jax.ShapeDtypeStruct((NP, 1), jnp.float32),
            jax.ShapeDtypeStruct((NP, 16), jnp.float32),
            jax.ShapeDtypeStruct((NP, 16), jnp.float32),
            jax.ShapeDtypeStruct((NP, 16), jnp.float32),
            jax.ShapeDtypeStruct((NP, 16), jnp.float32),
        ],
    )(x, w, deg2, deg2)


def _mid_body(ag0_ref, ag1_ref, ag2_ref, ag3_ref, q0_ref, q1_ref, q2_ref,
              q3_ref, dinv_ref, b1_ref, w2_ref, h2a_ref, h2b_ref):
    dinv = dinv_ref[...]
    a = jnp.concatenate([ag0_ref[...] + q0_ref[...],
                         ag1_ref[...] + q1_ref[...],
                         ag2_ref[...] + q2_ref[...],
                         ag3_ref[...] + q3_ref[...]], axis=1)
    h = jnp.maximum(dinv * a + b1_ref[...], 0.0)
    p2 = jnp.dot(h, w2_ref[...], preferred_element_type=jnp.float32)
    hs2 = dinv * p2
    h2a_ref[...] = hs2[:, :DO // 2]
    h2b_ref[...] = hs2[:, DO // 2:]


def _mid_call(a1p0, a1p1, q0, q1, q2, q3, dinv, b1, w2):
    return pl.pallas_call(
        _mid_body,
        grid=(NB,),
        in_specs=[
            pl.BlockSpec((RB, 16), lambda i: (i, 0)),
            pl.BlockSpec((RB, 16), lambda i: (NB + i, 0)),
            pl.BlockSpec((RB, 16), lambda i: (i, 0)),
            pl.BlockSpec((RB, 16), lambda i: (NB + i, 0)),
            pl.BlockSpec((RB, 16), lambda i: (i, 0)),
            pl.BlockSpec((RB, 16), lambda i: (i, 0)),
            pl.BlockSpec((RB, 16), lambda i: (i, 0)),
            pl.BlockSpec((RB, 16), lambda i: (i, 0)),
            pl.BlockSpec((RB, 1), lambda i: (i, 0)),
            pl.BlockSpec((1, DH), lambda i: (0, 0)),
            pl.BlockSpec((DH, DO), lambda i: (0, 0)),
        ],
        out_specs=[
            pl.BlockSpec((RB, DO // 2), lambda i: (i, 0)),
            pl.BlockSpec((RB, DO // 2), lambda i: (i, 0)),
        ],
        out_shape=[
            jax.ShapeDtypeStruct((NP, DO // 2), jnp.float32),
            jax.ShapeDtypeStruct((NP, DO // 2), jnp.float32),
        ],
    )(a1p0, a1p0, a1p1, a1p1, q0, q1, q2, q3, dinv, b1, w2)


def _out_body(aga_ref, agb_ref, hsa_ref, hsb_ref, dinv_ref, b2_ref, o_ref):
    a = jnp.concatenate([aga_ref[...] + hsa_ref[...],
                         agb_ref[...] + hsb_ref[...]], axis=1)
    o_ref[...] = dinv_ref[...] * a + b2_ref[...]


def _out_call(agg2, hs2a, hs2b, dinv, b2):
    return pl.pallas_call(
        _out_body,
        grid=(NB,),
        in_specs=[
            pl.BlockSpec((RB, DO // 2), lambda i: (i, 0)),
            pl.BlockSpec((RB, DO // 2), lambda i: (NB + i, 0)),
            pl.BlockSpec((RB, DO // 2), lambda i: (i, 0)),
            pl.BlockSpec((RB, DO // 2), lambda i: (i, 0)),
            pl.BlockSpec((RB, 1), lambda i: (i, 0)),
            pl.BlockSpec((1, DO), lambda i: (0, 0)),
        ],
        out_specs=pl.BlockSpec((RB, DO), lambda i: (i, 0)),
        out_shape=jax.ShapeDtypeStruct((NN, DO), jnp.float32),
    )(agg2, agg2, hs2a, hs2b, dinv, b2)


def kernel(x, edge_index, W1, b1, W2, b2):
    src = edge_index[0].astype(jnp.int32)
    dst = edge_index[1].astype(jnp.int32)
    npad = EPAD - EE
    srcI = jnp.concatenate([src, jnp.zeros((npad,), jnp.int32)]).reshape(
        EROWS, LANE)
    dstI = jnp.concatenate([dst, jnp.full((npad,), NN, jnp.int32)]).reshape(
        EROWS, LANE)
    zeros16 = jnp.zeros((NP, 16), jnp.float32)
    zerosd = jnp.zeros((NP, DEGW), jnp.float32)
    ones128 = jnp.ones((LANE, DEGW), jnp.float32)

    deg2 = _deg_call(dstI, zerosd, ones128)
    dinv, q0, q1, q2, q3 = _mm1_call(x, W1, deg2)
    a1p0 = _aggq_call(q0, q1, srcI, dstI, zeros16)
    a1p1 = _aggq_call(q2, q3, srcI, dstI, zeros16)
    hs2a, hs2b = _mid_call(a1p0, a1p1, q0, q1, q2, q3, dinv,
                           b1.reshape(1, DH), W2)
    agg2 = _aggq_call(hs2a, hs2b, srcI, dstI, zeros16)
    return _out_call(agg2, hs2a, hs2b, dinv, b2.reshape(1, DO))


# speedup vs baseline: 1.5077x; 1.3032x over previous
import functools

import jax
import jax.numpy as jnp
from jax import lax
from jax.experimental import pallas as pl
from jax.experimental.pallas import tpu as pltpu
from jax.experimental.pallas import tpu_sc as plsc

NN = 50000
EE = 800000
DIN = 768
DH = 64
DO = 32

LANE = 128
EROWS = 6400
EPAD = EROWS * LANE
RB = 4096
NP = 53248
NCORE = 2
NSUB = 16
NB = NP // RB
IB = 16
OUTER = EROWS // NSUB // IB
DIB = 20
DOUTER = EROWS // (NCORE * NSUB) // DIB

_mesh = plsc.VectorSubcoreMesh(core_axis_name="c", subcore_axis_name="s")
_sc_params = pltpu.CompilerParams(use_tc_tiling_on_sc=False)


DEGW = 16


def _deg_body(dst_hbm, zeros_hbm, ones_hbm, out_hbm, dst_bufs, ones_buf, acc,
              isem, ssem):
    cid = lax.axis_index("c")
    sid = lax.axis_index("s")
    zr = NP // NSUB
    pltpu.sync_copy(zeros_hbm.at[pl.ds(sid * zr, zr)], acc.at[pl.ds(sid * zr, zr)])
    pltpu.sync_copy(ones_hbm, ones_buf)
    plsc.subcore_barrier()
    rows_per = EROWS // (NCORE * NSUB)
    base = cid * (EROWS // NCORE) + sid * rows_per
    pltpu.sync_copy(dst_hbm.at[pl.ds(base, DIB)], dst_bufs.at[pl.ds(0, DIB)])

    @pl.loop(0, DOUTER)
    def _(g):
        @pl.when(g >= 1)
        def _():
            @pl.loop(0, DIB)
            def _(i):
                pltpu.make_async_copy(ones_hbm, ones_buf, ssem).wait()

        @pl.when(g < DOUTER - 1)
        def _():
            pltpu.async_copy(
                dst_hbm.at[pl.ds(base + (g + 1) * DIB, DIB)],
                dst_bufs.at[pl.ds(((g + 1) % 2) * DIB, DIB)], isem)
        for j in range(DIB):
            pltpu.async_copy(ones_buf, acc.at[dst_bufs.at[(g % 2) * DIB + j]],
                             ssem, add=True)

        @pl.when(g < DOUTER - 1)
        def _():
            pltpu.make_async_copy(
                dst_hbm.at[pl.ds(base, DIB)],
                dst_bufs.at[pl.ds(0, DIB)], isem).wait()

    @pl.loop(0, DIB)
    def _(i):
        pltpu.make_async_copy(ones_hbm, ones_buf, ssem).wait()

    plsc.subcore_barrier()
    pltpu.sync_copy(acc.at[pl.ds(sid * zr, zr)],
                    out_hbm.at[pl.ds(cid * NP + sid * zr, zr), pl.ds(0, DEGW)])


_deg_call = pl.kernel(
    _deg_body,
    out_type=jax.ShapeDtypeStruct((2 * NP, 128), jnp.float32),
    mesh=_mesh,
    compiler_params=_sc_params,
    scratch_types=[
        pltpu.VMEM((2 * DIB, LANE), jnp.int32),
        pltpu.VMEM((LANE, DEGW), jnp.float32),
        pltpu.VMEM_SHARED((NP, DEGW), jnp.float32),
        pltpu.SemaphoreType.DMA,
        pltpu.SemaphoreType.DMA,
    ],
)


def _aggw_body(passes, q_hbm, src_hbm, dst_hbm, zeros_hbm, out_hbm,
               src_bufs, dst_bufs, rows, acc, tabspm,
               gsems, ssems, isem_s, isem_d):
    cid = lax.axis_index("c")
    sid = lax.axis_index("s")
    zr = NP // NSUB
    rows_per = EROWS // NSUB
    ebase = sid * rows_per

    def wait_scatter(b):
        pltpu.make_async_copy(q_hbm.at[pl.ds(0, LANE), pl.ds(0, 16)],
                              rows.at[b], ssems.at[b]).wait()

    def wait_gather(b):
        pltpu.make_async_copy(q_hbm.at[pl.ds(0, LANE), pl.ds(0, 16)],
                              rows.at[b], gsems.at[b]).wait()

    def start_gather(idx_row, b):
        pltpu.async_copy(tabspm.at[src_bufs.at[idx_row]], rows.at[b],
                         gsems.at[b])

    for tb, ob in passes:
        pltpu.sync_copy(zeros_hbm.at[pl.ds(sid * zr, zr)],
                        acc.at[pl.ds(sid * zr, zr)])
        pltpu.sync_copy(q_hbm.at[pl.ds(sid * zr, zr), pl.ds(tb + cid * 16, 16)],
                        tabspm.at[pl.ds(sid * zr, zr)])
        pltpu.sync_copy(src_hbm.at[pl.ds(ebase, IB)], src_bufs.at[pl.ds(0, IB)])
        pltpu.sync_copy(dst_hbm.at[pl.ds(ebase, IB)], dst_bufs.at[pl.ds(0, IB)])
        plsc.subcore_barrier()
        for k in range(3):
            start_gather(k, k)

        @pl.loop(0, OUTER)
        def _(g):
            gb = (g % 2) * IB
            nb = ((g + 1) % 2) * IB
            for j in range(IB):
                b = j % 4
                b3 = (j + 3) % 4
                wait_gather(b)
                pltpu.async_copy(rows.at[b], acc.at[dst_bufs.at[gb + j]],
                                 ssems.at[b], add=True)
                if j == 1:
                    @pl.when(g < OUTER - 1)
                    def _():
                        pltpu.async_copy(
                            src_hbm.at[pl.ds(ebase + (g + 1) * IB, IB)],
                            src_bufs.at[pl.ds(nb, IB)], isem_s)
                        pltpu.async_copy(
                            dst_hbm.at[pl.ds(ebase + (g + 1) * IB, IB)],
                            dst_bufs.at[pl.ds(nb, IB)], isem_d)
                if j < IB - 3:
                    if j == 0:
                        @pl.when(g >= 1)
                        def _():
                            wait_scatter(b3)
                    else:
                        wait_scatter(b3)
                    start_gather(gb + j + 3, b3)
                else:
                    if j == IB - 3:
                        @pl.when(g < OUTER - 1)
                        def _():
                            pltpu.make_async_copy(src_hbm.at[pl.ds(0, IB)],
                                                  src_bufs.at[pl.ds(nb, IB)],
                                                  isem_s).wait()
                            pltpu.make_async_copy(dst_hbm.at[pl.ds(0, IB)],
                                                  dst_bufs.at[pl.ds(nb, IB)],
                                                  isem_d).wait()

                    @pl.when(g < OUTER - 1)
                    def _():
                        wait_scatter(b3)
                        start_gather(nb + j - (IB - 3), b3)

        for k in range(4):
            wait_scatter(k)
        plsc.subcore_barrier()
        pltpu.sync_copy(acc.at[pl.ds(sid * zr, zr)],
                        out_hbm.at[pl.ds(sid * zr, zr),
                                   pl.ds(ob + cid * 16, 16)])
        plsc.subcore_barrier()


def _make_aggw(passes):
    return pl.kernel(
        functools.partial(_aggw_body, passes),
        out_type=jax.ShapeDtypeStruct((NP, 128), jnp.float32),
        mesh=_mesh,
        compiler_params=_sc_params,
        scratch_types=[
            pltpu.VMEM((2 * IB, LANE), jnp.int32),
            pltpu.VMEM((2 * IB, LANE), jnp.int32),
            pltpu.VMEM((4, LANE, 16), jnp.float32),
            pltpu.VMEM_SHARED((NP, 16), jnp.float32),
            pltpu.VMEM_SHARED((NP, 16), jnp.float32),
            pltpu.SemaphoreType.DMA((4,)),
            pltpu.SemaphoreType.DMA((4,)),
            pltpu.SemaphoreType.DMA,
            pltpu.SemaphoreType.DMA,
        ],
    )


_agg1_call = _make_aggw([(0, 0), (32, 32)])
_agg2_call = _make_aggw([(0, 0)])


def _mm1_body(x_ref, w_ref, d0_ref, d1_ref, q_ref):
    deg = d0_ref[:, :1] + d1_ref[:, :1] + 1.0
    dinv = lax.rsqrt(deg)
    hs = jnp.dot(x_ref[...], w_ref[...],
                 preferred_element_type=jnp.float32) * dinv
    pad = jnp.zeros((RB, 128 - DH - 1), jnp.float32)
    q_ref[...] = jnp.concatenate([hs, dinv, pad], axis=1)


def _mm1_call(x, w, deg2):
    return pl.pallas_call(
        _mm1_body,
        grid=(NB,),
        in_specs=[
            pl.BlockSpec((RB, DIN), lambda i: (i, 0)),
            pl.BlockSpec((DIN, DH), lambda i: (0, 0)),
            pl.BlockSpec((RB, 128), lambda i: (i, 0)),
            pl.BlockSpec((RB, 128), lambda i: (NB + i, 0)),
        ],
        out_specs=pl.BlockSpec((RB, 128), lambda i: (i, 0)),
        out_shape=jax.ShapeDtypeStruct((NP, 128), jnp.float32),
    )(x, w, deg2, deg2)


def _mid_body(a1_ref, q1_ref, b1_ref, w2_ref, q2_ref):
    dinv = q1_ref[:, DH:DH + 1]
    h = jnp.maximum(
        dinv * (a1_ref[:, :DH] + q1_ref[:, :DH]) + b1_ref[...], 0.0)
    hs2 = dinv * jnp.dot(h, w2_ref[...], preferred_element_type=jnp.float32)
    pad = jnp.zeros((RB, 128 - DO - 1), jnp.float32)
    q2_ref[...] = jnp.concatenate([hs2, dinv, pad], axis=1)


def _mid_call(a1, q1, b1, w2):
    return pl.pallas_call(
        _mid_body,
        grid=(NB,),
        in_specs=[
            pl.BlockSpec((RB, 128), lambda i: (i, 0)),
            pl.BlockSpec((RB, 128), lambda i: (i, 0)),
            pl.BlockSpec((1, DH), lambda i: (0, 0)),
            pl.BlockSpec((DH, DO), lambda i: (0, 0)),
        ],
        out_specs=pl.BlockSpec((RB, 128), lambda i: (i, 0)),
        out_shape=jax.ShapeDtypeStruct((NP, 128), jnp.float32),
    )(a1, q1, b1, w2)


def _out_body(a2_ref, q2_ref, b2_ref, o_ref):
    dinv = q2_ref[:, DO:DO + 1]
    o_ref[...] = dinv * (a2_ref[:, :DO] + q2_ref[:, :DO]) + b2_ref[...]


def _out_call(a2, q2, b2):
    return pl.pallas_call(
        _out_body,
        grid=(NB,),
        in_specs=[
            pl.BlockSpec((RB, 128), lambda i: (i, 0)),
            pl.BlockSpec((RB, 128), lambda i: (i, 0)),
            pl.BlockSpec((1, DO), lambda i: (0, 0)),
        ],
        out_specs=pl.BlockSpec((RB, DO), lambda i: (i, 0)),
        out_shape=jax.ShapeDtypeStruct((NN, DO), jnp.float32),
    )(a2, q2, b2)


def kernel(x, edge_index, W1, b1, W2, b2):
    src = edge_index[0].astype(jnp.int32)
    dst = edge_index[1].astype(jnp.int32)
    npad = EPAD - EE
    srcI = jnp.concatenate([src, jnp.zeros((npad,), jnp.int32)]).reshape(
        EROWS, LANE)
    dstI = jnp.concatenate([dst, jnp.full((npad,), NN, jnp.int32)]).reshape(
        EROWS, LANE)
    zeros16 = jnp.zeros((NP, 16), jnp.float32)
    zerosd = jnp.zeros((NP, DEGW), jnp.float32)
    ones128 = jnp.ones((LANE, DEGW), jnp.float32)

    deg2 = _deg_call(dstI, zerosd, ones128)
    q1 = _mm1_call(x, W1, deg2)
    a1 = _agg1_call(q1, srcI, dstI, zeros16)
    q2 = _mid_call(a1, q1, b1.reshape(1, DH), W2)
    a2 = _agg2_call(q2, srcI, dstI, zeros16)
    return _out_call(a2, q2, b2.reshape(1, DO))
